# Initial kernel scaffold; baseline (speedup 1.0000x reference)
import jax
import jax.numpy as jnp
from jax.experimental import pallas as pl


def kernel(x, edge_index, W1_rel, b1, W1_root, W2_rel, b2, W2_root, W3_rel, b3, W3_root):
    raise NotImplementedError("write your pallas kernel here")


# speedup vs baseline: 4.2510x; 4.2510x over previous
import functools

import jax
import jax.numpy as jnp
from jax import lax
from jax.experimental import pallas as pl
from jax.experimental.pallas import tpu as pltpu
from jax.experimental.pallas import tpu_sc as plsc

N_NODES = 10000
N_EDGES = 160000
D_IN = 256
D_H1 = 512
D_H2 = 256

NC = 2
NS = 16
CHUNK = 128

EP = 163840
N_ACC = 10112
NCH16 = EP // (NS * CHUNK)
NCH32 = EP // (NC * NS * CHUNK)

def _make_segsum(feat, nch, per_core_edges):
    mesh = plsc.VectorSubcoreMesh(
        core_axis_name="c", subcore_axis_name="s",
        num_cores=NC, num_subcores=NS)

    G = 8
    ng = nch // G

    @functools.partial(
        pl.kernel,
        mesh=mesh,
        out_type=jax.ShapeDtypeStruct((NC, N_ACC, feat), jnp.float32),
        scratch_types=[
            pltpu.VMEM_SHARED((N_ACC, feat), jnp.float32),
            pltpu.VMEM((G, CHUNK), jnp.int32),
            pltpu.VMEM((G, CHUNK), jnp.int32),
            pltpu.VMEM((CHUNK, feat), jnp.float32),
            pltpu.VMEM((CHUNK, feat), jnp.float32),
            pltpu.SemaphoreType.DMA,
            pltpu.SemaphoreType.DMA,
        ],
    )
    def seg(tbl, srcr, dstr, zr, out, acc, srcv, dstv, rb0, rb1, sm0, sm1):
        c = lax.axis_index("c")
        s = lax.axis_index("s")
        zrows = N_ACC // NS
        pltpu.sync_copy(zr.at[pl.ds(s * zrows, zrows)],
                        acc.at[pl.ds(s * zrows, zrows)])
        eslice = c * NS + s if per_core_edges else s
        srcrt = srcr.at[eslice]
        dstrt = dstr.at[eslice]
        plsc.subcore_barrier()

        tblc = tbl if per_core_edges else tbl.at[c]
        bufs = (rb0, rb1)
        sems = (sm0, sm1)

        @pl.loop(0, ng)
        def _(g):
            pltpu.sync_copy(srcrt.at[pl.ds(g * G, G)], srcv)
            pltpu.sync_copy(dstrt.at[pl.ds(g * G, G)], dstv)
            pltpu.async_copy(tblc.at[srcv.at[0]], rb0, sm0)
            pltpu.async_copy(tblc.at[srcv.at[1]], rb1, sm1)
            for k in range(G):
                b = k % 2
                pltpu.make_async_copy(tblc.at[srcv.at[k]], bufs[b],
                                      sems[b]).wait()
                pltpu.sync_copy(bufs[b], acc.at[dstv.at[k]], add=True)
                if k + 2 < G:
                    pltpu.async_copy(tblc.at[srcv.at[k + 2]], bufs[b],
                                     sems[b])

        plsc.subcore_barrier()
        pltpu.sync_copy(acc.at[pl.ds(s * zrows, zrows)],
                        out.at[c].at[pl.ds(s * zrows, zrows)])

    return seg


@functools.cache
def _segsum_feat_kernel():
    return _make_segsum(128, NCH16, per_core_edges=False)


@functools.cache
def _segsum_edge_kernel():
    return _make_segsum(128, NCH32, per_core_edges=True)


def _segsum_feat(*args):
    return _segsum_feat_kernel()(*args)


def _segsum_edge(*args):
    return _segsum_edge_kernel()(*args)

_R = 1000


def _b_body(aggh, x, w1r, w1s, b1, w2r, w2s, y2h, r2):
    agg = jnp.concatenate([aggh[0], aggh[1]], axis=-1)
    h = jnp.dot(agg, w1r[...], preferred_element_type=jnp.float32)
    h = h + jnp.dot(x[...], w1s[...], preferred_element_type=jnp.float32)
    x1 = jnp.maximum(h + b1[...], 0.0)
    y2 = jnp.dot(x1, w2r[...], preferred_element_type=jnp.float32)
    y2h[0] = y2[:, :128]
    y2h[1] = y2[:, 128:]
    r2[...] = jnp.dot(x1, w2s[...], preferred_element_type=jnp.float32)


def _c_body(aggh, r2, b2, w3r, w3s, y3p, r3p):
    x2 = jnp.concatenate([aggh[0], aggh[1]], axis=-1) + r2[...] + b2[...]
    y3p[...] = jnp.dot(x2, w3r[...], preferred_element_type=jnp.float32)
    r3p[...] = jnp.dot(x2, w3s[...], preferred_element_type=jnp.float32)


def _e_body(part, r3p, b3, out):
    l = part[0][:, :16] + part[1][:, :16] + r3p[...] + b3[...]
    l0 = l[:, 0:1]
    l1 = l[:, 1:2]
    m = jnp.maximum(l0, l1)
    e0 = jnp.exp(l0 - m)
    e1 = jnp.exp(l1 - m)
    inv = 1.0 / (e0 + e1)
    out[...] = jnp.concatenate([e0 * inv, e1 * inv], axis=1)


def _dense_b(agg1h, x, w1rT, w1sT, b1r, w2rT, w2sT):
    g = N_NODES // _R
    return pl.pallas_call(
        _b_body,
        grid=(g,),
        in_specs=[
            pl.BlockSpec((NC, _R, 128), lambda i: (0, i, 0)),
            pl.BlockSpec((_R, D_IN), lambda i: (i, 0)),
            pl.BlockSpec((D_IN, D_H1), lambda i: (0, 0)),
            pl.BlockSpec((D_IN, D_H1), lambda i: (0, 0)),
            pl.BlockSpec((1, D_H1), lambda i: (0, 0)),
            pl.BlockSpec((D_H1, D_H2), lambda i: (0, 0)),
            pl.BlockSpec((D_H1, D_H2), lambda i: (0, 0)),
        ],
        out_specs=[
            pl.BlockSpec((NC, _R, 128), lambda i: (0, i, 0)),
            pl.BlockSpec((_R, D_H2), lambda i: (i, 0)),
        ],
        out_shape=[
            jax.ShapeDtypeStruct((NC, N_NODES, 128), jnp.float32),
            jax.ShapeDtypeStruct((N_NODES, D_H2), jnp.float32),
        ],
        compiler_params=pltpu.CompilerParams(
            dimension_semantics=("parallel",)),
    )(agg1h, x, w1rT, w1sT, b1r, w2rT, w2sT)


def _dense_c(agg2h, r2, b2r, w3rP, w3sP):
    g = N_NODES // _R
    return pl.pallas_call(
        _c_body,
        grid=(g,),
        in_specs=[
            pl.BlockSpec((NC, _R, 128), lambda i: (0, i, 0)),
            pl.BlockSpec((_R, D_H2), lambda i: (i, 0)),
            pl.BlockSpec((1, D_H2), lambda i: (0, 0)),
            pl.BlockSpec((D_H2, 128), lambda i: (0, 0)),
            pl.BlockSpec((D_H2, 16), lambda i: (0, 0)),
        ],
        out_specs=[
            pl.BlockSpec((_R, 128), lambda i: (i, 0)),
            pl.BlockSpec((_R, 16), lambda i: (i, 0)),
        ],
        out_shape=[
            jax.ShapeDtypeStruct((N_NODES, 128), jnp.float32),
            jax.ShapeDtypeStruct((N_NODES, 16), jnp.float32),
        ],
        compiler_params=pltpu.CompilerParams(
            dimension_semantics=("parallel",)),
    )(agg2h, r2, b2r, w3rP, w3sP)


def _dense_e(part, r3p, b3r):
    g = N_NODES // _R
    return pl.pallas_call(
        _e_body,
        grid=(g,),
        in_specs=[
            pl.BlockSpec((NC, _R, 128), lambda i: (0, i, 0)),
            pl.BlockSpec((_R, 16), lambda i: (i, 0)),
            pl.BlockSpec((1, 16), lambda i: (0, 0)),
        ],
        out_specs=pl.BlockSpec((_R, 2), lambda i: (i, 0)),
        out_shape=jax.ShapeDtypeStruct((N_NODES, 2), jnp.float32),
        compiler_params=pltpu.CompilerParams(
            dimension_semantics=("parallel",)),
    )(part, r3p, b3r)


def kernel(x, edge_index, W1_rel, b1, W1_root, W2_rel, b2, W2_root, W3_rel,
           b3, W3_root):
    src = edge_index[0].astype(jnp.int32)
    dst = edge_index[1].astype(jnp.int32)
    srcp = jnp.concatenate([src, jnp.zeros((EP - N_EDGES,), jnp.int32)])
    dstp = jnp.concatenate(
        [dst, jnp.full((EP - N_EDGES,), N_NODES, jnp.int32)])
    srcr16 = srcp.reshape(NS, NCH16, CHUNK)
    dstr16 = dstp.reshape(NS, NCH16, CHUNK)
    srcr32 = srcp.reshape(NC * NS, NCH32, CHUNK)
    dstr32 = dstp.reshape(NC * NS, NCH32, CHUNK)
    z128 = jnp.zeros((N_ACC, 128), jnp.float32)

    xh = jnp.stack([x[:, :128], x[:, 128:]])
    w1rT = W1_rel.T
    w1sT = W1_root.T
    w2rT = W2_rel.T
    w2sT = W2_root.T
    w3rP = jnp.pad(W3_rel.T, ((0, 0), (0, 126)))
    w3sP = jnp.pad(W3_root.T, ((0, 0), (0, 14)))
    b1r = b1.reshape(1, D_H1)
    b2r = b2.reshape(1, D_H2)
    b3r = jnp.pad(b3, (0, 14)).reshape(1, 16)

    agg1h = _segsum_feat(xh, srcr16, dstr16, z128)
    y2h, r2 = _dense_b(agg1h, x, w1rT, w1sT, b1r, w2rT, w2sT)
    agg2h = _segsum_feat(y2h, srcr16, dstr16, z128)
    y3p, r3p = _dense_c(agg2h, r2, b2r, w3rP, w3sP)
    part = _segsum_edge(y3p, srcr32, dstr32, z128)
    return _dense_e(part, r3p, b3r)


# speedup vs baseline: 5.2550x; 1.2362x over previous
import functools

import jax
import jax.numpy as jnp
from jax import lax
from jax.experimental import pallas as pl
from jax.experimental.pallas import tpu as pltpu
from jax.experimental.pallas import tpu_sc as plsc

N_NODES = 10000
N_EDGES = 160000
D_IN = 256
D_H1 = 512
D_H2 = 256

NC = 2
NS = 16
CHUNK = 128

EP = 163840
N_ACC = 10112
NCH16 = EP // (NS * CHUNK)
NCH32 = EP // (NC * NS * CHUNK)

def _make_segsum(feat, nch, per_core_edges, tc_tiling=None):
    mesh = plsc.VectorSubcoreMesh(
        core_axis_name="c", subcore_axis_name="s",
        num_cores=NC, num_subcores=NS)

    G = 8
    ng = nch // G
    cparams = (None if tc_tiling is None else
               pltpu.CompilerParams(use_tc_tiling_on_sc=tc_tiling))

    @functools.partial(
        pl.kernel,
        mesh=mesh,
        compiler_params=cparams,
        out_type=jax.ShapeDtypeStruct((NC, N_ACC, feat), jnp.float32),
        scratch_types=[
            pltpu.VMEM_SHARED((N_ACC, feat), jnp.float32),
            pltpu.VMEM((G, CHUNK), jnp.int32),
            pltpu.VMEM((G, CHUNK), jnp.int32),
            pltpu.VMEM((CHUNK, feat), jnp.float32),
            pltpu.VMEM((CHUNK, feat), jnp.float32),
            pltpu.SemaphoreType.DMA,
            pltpu.SemaphoreType.DMA,
        ],
    )
    def seg(tbl, srcr, dstr, zr, out, acc, srcv, dstv, rb0, rb1, sm0, sm1):
        c = lax.axis_index("c")
        s = lax.axis_index("s")
        zrows = N_ACC // NS
        pltpu.sync_copy(zr.at[pl.ds(s * zrows, zrows)],
                        acc.at[pl.ds(s * zrows, zrows)])
        eslice = c * NS + s if per_core_edges else s
        srcrt = srcr.at[eslice]
        dstrt = dstr.at[eslice]
        plsc.subcore_barrier()

        tblc = tbl if per_core_edges else tbl.at[c]
        bufs = (rb0, rb1)
        sems = (sm0, sm1)

        @pl.loop(0, ng)
        def _(g):
            pltpu.sync_copy(srcrt.at[pl.ds(g * G, G)], srcv)
            pltpu.sync_copy(dstrt.at[pl.ds(g * G, G)], dstv)
            pltpu.async_copy(tblc.at[srcv.at[0]], rb0, sm0)
            pltpu.async_copy(tblc.at[srcv.at[1]], rb1, sm1)
            for k in range(G):
                b = k % 2
                pltpu.make_async_copy(tblc.at[srcv.at[k]], bufs[b],
                                      sems[b]).wait()
                pltpu.sync_copy(bufs[b], acc.at[dstv.at[k]], add=True)
                if k + 2 < G:
                    pltpu.async_copy(tblc.at[srcv.at[k + 2]], bufs[b],
                                     sems[b])

        plsc.subcore_barrier()
        pltpu.sync_copy(acc.at[pl.ds(s * zrows, zrows)],
                        out.at[c].at[pl.ds(s * zrows, zrows)])

    return seg


@functools.cache
def _segsum_feat_kernel():
    return _make_segsum(128, NCH16, per_core_edges=False)


@functools.cache
def _segsum_edge_kernel():
    mesh = plsc.VectorSubcoreMesh(
        core_axis_name="c", subcore_axis_name="s",
        num_cores=NC, num_subcores=NS)
    G = 8
    ng = NCH32 // G

    @functools.partial(
        pl.kernel,
        mesh=mesh,
        compiler_params=pltpu.CompilerParams(use_tc_tiling_on_sc=False),
        out_type=jax.ShapeDtypeStruct((4 * N_ACC,), jnp.float32),
        scratch_types=[
            pltpu.VMEM_SHARED((2, N_ACC), jnp.float32),
            pltpu.VMEM((G, CHUNK), jnp.int32),
            pltpu.VMEM((G, CHUNK), jnp.int32),
            pltpu.VMEM((2, 2, CHUNK), jnp.float32),
            pltpu.SemaphoreType.DMA,
            pltpu.SemaphoreType.DMA,
        ],
    )
    def seg(tbl0, tbl1, srcr, dstr, zr, out, acc, srcv, dstv, vb, sm0, sm1):
        c = lax.axis_index("c")
        s = lax.axis_index("s")
        zrows = N_ACC // NS
        pltpu.sync_copy(zr.at[pl.ds(s * zrows, zrows)],
                        acc.at[0].at[pl.ds(s * zrows, zrows)])
        pltpu.sync_copy(zr.at[pl.ds(s * zrows, zrows)],
                        acc.at[1].at[pl.ds(s * zrows, zrows)])
        eslice = c * NS + s
        srcrt = srcr.at[eslice]
        dstrt = dstr.at[eslice]
        plsc.subcore_barrier()

        tbls = (tbl0, tbl1)
        sems = (sm0, sm1)

        @pl.loop(0, ng)
        def _(g):
            pltpu.sync_copy(srcrt.at[pl.ds(g * G, G)], srcv)
            pltpu.sync_copy(dstrt.at[pl.ds(g * G, G)], dstv)
            for col in range(2):
                pltpu.async_copy(tbls[col].at[srcv.at[0]],
                                 vb.at[0].at[col], sems[col])
            for k in range(G):
                b = k % 2
                for col in range(2):
                    pltpu.make_async_copy(tbls[col].at[srcv.at[k]],
                                          vb.at[b].at[col], sems[col]).wait()
                    if k + 1 < G:
                        pltpu.async_copy(tbls[col].at[srcv.at[k + 1]],
                                         vb.at[1 - b].at[col], sems[col])
                for col in range(2):
                    pltpu.sync_copy(vb.at[b].at[col],
                                    acc.at[col].at[dstv.at[k]], add=True)

        plsc.subcore_barrier()
        for col in range(2):
            base = c * (2 * N_ACC) + col * N_ACC + s * zrows
            pltpu.sync_copy(acc.at[col].at[pl.ds(s * zrows, zrows)],
                            out.at[pl.ds(base, zrows)])

    return seg


def _segsum_feat(*args):
    return _segsum_feat_kernel()(*args)


def _segsum_edge(*args):
    return _segsum_edge_kernel()(*args)

_R = 1024
_G = -(-N_NODES // _R)


def _b_body(aggh, x, w1r, w1s, b1, w2r, w2s, y2h, r2):
    agg = jnp.concatenate([aggh[0], aggh[1]], axis=-1)
    h = jnp.dot(agg, w1r[...], preferred_element_type=jnp.float32)
    h = h + jnp.dot(x[...], w1s[...], preferred_element_type=jnp.float32)
    x1 = jnp.maximum(h + b1[...], 0.0)
    y2 = jnp.dot(x1, w2r[...], preferred_element_type=jnp.float32)
    y2h[0] = y2[:, :128]
    y2h[1] = y2[:, 128:]
    r2[...] = jnp.dot(x1, w2s[...], preferred_element_type=jnp.float32)


def _c_body(aggh, r2, b2, w3r, w3s, y3c0, y3c1, r3c0, r3c1):
    x2 = jnp.concatenate([aggh[0], aggh[1]], axis=-1) + r2[...] + b2[...]
    y3 = jnp.dot(x2, w3r[...], preferred_element_type=jnp.float32)
    r3 = jnp.dot(x2, w3s[...], preferred_element_type=jnp.float32)
    y3c0[...] = y3[:, 0]
    y3c1[...] = y3[:, 1]
    r3c0[...] = r3[:, 0]
    r3c1[...] = r3[:, 1]


def _e_body(partf, r3c0, r3c1, b3, o0, o1):
    l0 = (partf[0:N_NODES] + partf[2 * N_ACC:2 * N_ACC + N_NODES]
          + r3c0[...] + b3[0:1])
    l1 = (partf[N_ACC:N_ACC + N_NODES]
          + partf[3 * N_ACC:3 * N_ACC + N_NODES] + r3c1[...] + b3[1:2])
    m = jnp.maximum(l0, l1)
    e0 = jnp.exp(l0 - m)
    e1 = jnp.exp(l1 - m)
    inv = 1.0 / (e0 + e1)
    o0[...] = e0 * inv
    o1[...] = e1 * inv


def _dense_b(agg1h, x, w1rT, w1sT, b1r, w2rT, w2sT):
    return pl.pallas_call(
        _b_body,
        grid=(_G,),
        in_specs=[
            pl.BlockSpec((NC, _R, 128), lambda i: (0, i, 0)),
            pl.BlockSpec((_R, D_IN), lambda i: (i, 0)),
            pl.BlockSpec((D_IN, D_H1), lambda i: (0, 0)),
            pl.BlockSpec((D_IN, D_H1), lambda i: (0, 0)),
            pl.BlockSpec((1, D_H1), lambda i: (0, 0)),
            pl.BlockSpec((D_H1, D_H2), lambda i: (0, 0)),
            pl.BlockSpec((D_H1, D_H2), lambda i: (0, 0)),
        ],
        out_specs=[
            pl.BlockSpec((NC, _R, 128), lambda i: (0, i, 0)),
            pl.BlockSpec((_R, D_H2), lambda i: (i, 0)),
        ],
        out_shape=[
            jax.ShapeDtypeStruct((NC, N_NODES, 128), jnp.float32),
            jax.ShapeDtypeStruct((N_NODES, D_H2), jnp.float32),
        ],
        compiler_params=pltpu.CompilerParams(
            dimension_semantics=("parallel",)),
    )(agg1h, x, w1rT, w1sT, b1r, w2rT, w2sT)


def _dense_c(agg2h, r2, b2r, w3rT, w3sT):
    return pl.pallas_call(
        _c_body,
        grid=(_G,),
        in_specs=[
            pl.BlockSpec((NC, _R, 128), lambda i: (0, i, 0)),
            pl.BlockSpec((_R, D_H2), lambda i: (i, 0)),
            pl.BlockSpec((1, D_H2), lambda i: (0, 0)),
            pl.BlockSpec((D_H2, 2), lambda i: (0, 0)),
            pl.BlockSpec((D_H2, 2), lambda i: (0, 0)),
        ],
        out_specs=[
            pl.BlockSpec((_R,), lambda i: (i,)),
            pl.BlockSpec((_R,), lambda i: (i,)),
            pl.BlockSpec((_R,), lambda i: (i,)),
            pl.BlockSpec((_R,), lambda i: (i,)),
        ],
        out_shape=[
            jax.ShapeDtypeStruct((N_NODES,), jnp.float32),
            jax.ShapeDtypeStruct((N_NODES,), jnp.float32),
            jax.ShapeDtypeStruct((N_NODES,), jnp.float32),
            jax.ShapeDtypeStruct((N_NODES,), jnp.float32),
        ],
        compiler_params=pltpu.CompilerParams(
            dimension_semantics=("parallel",)),
    )(agg2h, r2, b2r, w3rT, w3sT)


def _dense_e(partf, r3c0, r3c1, b3):
    return pl.pallas_call(
        _e_body,
        grid=(1,),
        in_specs=[
            pl.BlockSpec((4 * N_ACC,), lambda i: (0,)),
            pl.BlockSpec((N_NODES,), lambda i: (0,)),
            pl.BlockSpec((N_NODES,), lambda i: (0,)),
            pl.BlockSpec((2,), lambda i: (0,)),
        ],
        out_specs=[
            pl.BlockSpec((N_NODES,), lambda i: (0,)),
            pl.BlockSpec((N_NODES,), lambda i: (0,)),
        ],
        out_shape=[
            jax.ShapeDtypeStruct((N_NODES,), jnp.float32),
            jax.ShapeDtypeStruct((N_NODES,), jnp.float32),
        ],
        compiler_params=pltpu.CompilerParams(
            dimension_semantics=("arbitrary",)),
    )(partf, r3c0, r3c1, b3)


def kernel(x, edge_index, W1_rel, b1, W1_root, W2_rel, b2, W2_root, W3_rel,
           b3, W3_root):
    src = edge_index[0].astype(jnp.int32)
    dst = edge_index[1].astype(jnp.int32)
    srcp = jnp.concatenate([src, jnp.zeros((EP - N_EDGES,), jnp.int32)])
    dstp = jnp.concatenate(
        [dst, jnp.full((EP - N_EDGES,), N_NODES, jnp.int32)])
    srcr16 = srcp.reshape(NS, NCH16, CHUNK)
    dstr16 = dstp.reshape(NS, NCH16, CHUNK)
    srcr32 = srcp.reshape(NC * NS, NCH32, CHUNK)
    dstr32 = dstp.reshape(NC * NS, NCH32, CHUNK)
    z128 = jnp.zeros((N_ACC, 128), jnp.float32)
    z1 = jnp.zeros((N_ACC,), jnp.float32)

    xh = jnp.stack([x[:, :128], x[:, 128:]])
    w1rT = W1_rel.T
    w1sT = W1_root.T
    w2rT = W2_rel.T
    w2sT = W2_root.T
    w3rT = W3_rel.T
    w3sT = W3_root.T
    b1r = b1.reshape(1, D_H1)
    b2r = b2.reshape(1, D_H2)

    agg1h = _segsum_feat(xh, srcr16, dstr16, z128)
    y2h, r2 = _dense_b(agg1h, x, w1rT, w1sT, b1r, w2rT, w2sT)
    agg2h = _segsum_feat(y2h, srcr16, dstr16, z128)
    y3c0, y3c1, r3c0, r3c1 = _dense_c(agg2h, r2, b2r, w3rT, w3sT)
    partf = _segsum_edge(y3c0, y3c1, srcr32, dstr32, z1)
    o0, o1 = _dense_e(partf, r3c0, r3c1, b3)
    return jnp.stack([o0, o1], axis=1)


# speedup vs baseline: 5.3827x; 1.0243x over previous
import functools

import jax
import jax.numpy as jnp
from jax import lax
from jax.experimental import pallas as pl
from jax.experimental.pallas import tpu as pltpu
from jax.experimental.pallas import tpu_sc as plsc

N_NODES = 10000
N_EDGES = 160000
D_IN = 256
D_H1 = 512
D_H2 = 256

NC = 2
NS = 16
CHUNK = 128

EP = 163840
N_ACC = 10112
NCH16 = EP // (NS * CHUNK)
NCH32 = EP // (NC * NS * CHUNK)

def _make_segsum(feat, nch, per_core_edges, tc_tiling=None):
    mesh = plsc.VectorSubcoreMesh(
        core_axis_name="c", subcore_axis_name="s",
        num_cores=NC, num_subcores=NS)

    G = 8
    ng = nch // G
    cparams = (None if tc_tiling is None else
               pltpu.CompilerParams(use_tc_tiling_on_sc=tc_tiling))

    @functools.partial(
        pl.kernel,
        mesh=mesh,
        compiler_params=cparams,
        out_type=jax.ShapeDtypeStruct((NC, N_ACC, feat), jnp.float32),
        scratch_types=[
            pltpu.VMEM_SHARED((N_ACC, feat), jnp.float32),
            pltpu.VMEM((2, G, CHUNK), jnp.int32),
            pltpu.VMEM((2, G, CHUNK), jnp.int32),
            pltpu.VMEM((CHUNK, feat), jnp.float32),
            pltpu.VMEM((CHUNK, feat), jnp.float32),
            pltpu.SemaphoreType.DMA,
            pltpu.SemaphoreType.DMA,
            pltpu.SemaphoreType.DMA,
            pltpu.SemaphoreType.DMA,
        ],
    )
    def seg(tbl, srcr, dstr, zr, out, acc, sv, dv, rb0, rb1,
            sm0, sm1, im0, im1):
        c = lax.axis_index("c")
        s = lax.axis_index("s")
        zrows = N_ACC // NS
        pltpu.sync_copy(zr.at[pl.ds(s * zrows, zrows)],
                        acc.at[pl.ds(s * zrows, zrows)])
        eslice = c * NS + s if per_core_edges else s
        srcrt = srcr.at[eslice]
        dstrt = dstr.at[eslice]

        tblc = tbl if per_core_edges else tbl.at[c]
        bufs = (rb0, rb1)
        sems = (sm0, sm1)
        isems = (im0, im1)

        pltpu.async_copy(srcrt.at[pl.ds(0, G)], sv.at[0], im0)
        pltpu.async_copy(dstrt.at[pl.ds(0, G)], dv.at[0], im0)
        plsc.subcore_barrier()

        @pl.loop(0, ng // 2)
        def _(h):
            for q in range(2):
                g = 2 * h + q
                svq = sv.at[q]
                dvq = dv.at[q]
                pltpu.make_async_copy(srcrt.at[pl.ds(0, G)], svq,
                                      isems[q]).wait()
                pltpu.make_async_copy(dstrt.at[pl.ds(0, G)], dvq,
                                      isems[q]).wait()

                @pl.when(g + 1 < ng)
                def _pref():
                    pltpu.async_copy(srcrt.at[pl.ds((g + 1) * G, G)],
                                     sv.at[1 - q], isems[1 - q])
                    pltpu.async_copy(dstrt.at[pl.ds((g + 1) * G, G)],
                                     dv.at[1 - q], isems[1 - q])

                pltpu.async_copy(tblc.at[svq.at[0]], rb0, sm0)
                pltpu.async_copy(tblc.at[svq.at[1]], rb1, sm1)
                for k in range(G):
                    b = k % 2
                    pltpu.make_async_copy(tblc.at[svq.at[k]], bufs[b],
                                          sems[b]).wait()
                    if k + 2 < G:
                        pltpu.async_copy(tblc.at[svq.at[k + 2]], bufs[b],
                                         sems[b])
                    pltpu.sync_copy(bufs[b], acc.at[dvq.at[k]], add=True)

        plsc.subcore_barrier()
        pltpu.sync_copy(acc.at[pl.ds(s * zrows, zrows)],
                        out.at[c].at[pl.ds(s * zrows, zrows)])

    return seg


@functools.cache
def _segsum_feat_kernel():
    return _make_segsum(128, NCH16, per_core_edges=False)


@functools.cache
def _segsum_edge_kernel():
    mesh = plsc.VectorSubcoreMesh(
        core_axis_name="c", subcore_axis_name="s",
        num_cores=NC, num_subcores=NS)
    G = 8
    ng = NCH32 // G

    @functools.partial(
        pl.kernel,
        mesh=mesh,
        compiler_params=pltpu.CompilerParams(use_tc_tiling_on_sc=False),
        out_type=jax.ShapeDtypeStruct((4 * N_ACC,), jnp.float32),
        scratch_types=[
            pltpu.VMEM_SHARED((2, N_ACC), jnp.float32),
            pltpu.VMEM((G, CHUNK), jnp.int32),
            pltpu.VMEM((G, CHUNK), jnp.int32),
            pltpu.VMEM((2, 2, CHUNK), jnp.float32),
            pltpu.SemaphoreType.DMA,
            pltpu.SemaphoreType.DMA,
        ],
    )
    def seg(tbl0, tbl1, srcr, dstr, zr, out, acc, srcv, dstv, vb, sm0, sm1):
        c = lax.axis_index("c")
        s = lax.axis_index("s")
        zrows = N_ACC // NS
        pltpu.sync_copy(zr.at[pl.ds(s * zrows, zrows)],
                        acc.at[0].at[pl.ds(s * zrows, zrows)])
        pltpu.sync_copy(zr.at[pl.ds(s * zrows, zrows)],
                        acc.at[1].at[pl.ds(s * zrows, zrows)])
        eslice = c * NS + s
        srcrt = srcr.at[eslice]
        dstrt = dstr.at[eslice]
        plsc.subcore_barrier()

        tbls = (tbl0, tbl1)
        sems = (sm0, sm1)

        @pl.loop(0, ng)
        def _(g):
            pltpu.sync_copy(srcrt.at[pl.ds(g * G, G)], srcv)
            pltpu.sync_copy(dstrt.at[pl.ds(g * G, G)], dstv)
            for col in range(2):
                pltpu.async_copy(tbls[col].at[srcv.at[0]],
                                 vb.at[0].at[col], sems[col])
            for k in range(G):
                b = k % 2
                for col in range(2):
                    pltpu.make_async_copy(tbls[col].at[srcv.at[k]],
                                          vb.at[b].at[col], sems[col]).wait()
                    if k + 1 < G:
                        pltpu.async_copy(tbls[col].at[srcv.at[k + 1]],
                                         vb.at[1 - b].at[col], sems[col])
                for col in range(2):
                    pltpu.sync_copy(vb.at[b].at[col],
                                    acc.at[col].at[dstv.at[k]], add=True)

        plsc.subcore_barrier()
        for col in range(2):
            base = c * (2 * N_ACC) + col * N_ACC + s * zrows
            pltpu.sync_copy(acc.at[col].at[pl.ds(s * zrows, zrows)],
                            out.at[pl.ds(base, zrows)])

    return seg


def _segsum_feat(*args):
    return _segsum_feat_kernel()(*args)


def _segsum_edge(*args):
    return _segsum_edge_kernel()(*args)

_R = 1024
_G = -(-N_NODES // _R)


def _b_body(aggh, x, w1r, w1s, b1, w2r, w2s, y2h, r2):
    agg = jnp.concatenate([aggh[0], aggh[1]], axis=-1)
    h = jnp.dot(agg, w1r[...], preferred_element_type=jnp.float32)
    h = h + jnp.dot(x[...], w1s[...], preferred_element_type=jnp.float32)
    x1 = jnp.maximum(h + b1[...], 0.0)
    y2 = jnp.dot(x1, w2r[...], preferred_element_type=jnp.float32)
    y2h[0] = y2[:, :128]
    y2h[1] = y2[:, 128:]
    r2[...] = jnp.dot(x1, w2s[...], preferred_element_type=jnp.float32)


def _c_body(aggh, r2, b2, w3r, w3s, y3c0, y3c1, r3c0, r3c1):
    x2 = jnp.concatenate([aggh[0], aggh[1]], axis=-1) + r2[...] + b2[...]
    y3 = jnp.dot(x2, w3r[...], preferred_element_type=jnp.float32)
    r3 = jnp.dot(x2, w3s[...], preferred_element_type=jnp.float32)
    y3c0[...] = y3[:, 0]
    y3c1[...] = y3[:, 1]
    r3c0[...] = r3[:, 0]
    r3c1[...] = r3[:, 1]


def _e_body(partf, r3c0, r3c1, b3, o0, o1):
    l0 = (partf[0:N_NODES] + partf[2 * N_ACC:2 * N_ACC + N_NODES]
          + r3c0[...] + b3[0:1])
    l1 = (partf[N_ACC:N_ACC + N_NODES]
          + partf[3 * N_ACC:3 * N_ACC + N_NODES] + r3c1[...] + b3[1:2])
    m = jnp.maximum(l0, l1)
    e0 = jnp.exp(l0 - m)
    e1 = jnp.exp(l1 - m)
    inv = 1.0 / (e0 + e1)
    o0[...] = e0 * inv
    o1[...] = e1 * inv


def _dense_b(agg1h, x, w1rT, w1sT, b1r, w2rT, w2sT):
    return pl.pallas_call(
        _b_body,
        grid=(_G,),
        in_specs=[
            pl.BlockSpec((NC, _R, 128), lambda i: (0, i, 0)),
            pl.BlockSpec((_R, D_IN), lambda i: (i, 0)),
            pl.BlockSpec((D_IN, D_H1), lambda i: (0, 0)),
            pl.BlockSpec((D_IN, D_H1), lambda i: (0, 0)),
            pl.BlockSpec((1, D_H1), lambda i: (0, 0)),
            pl.BlockSpec((D_H1, D_H2), lambda i: (0, 0)),
            pl.BlockSpec((D_H1, D_H2), lambda i: (0, 0)),
        ],
        out_specs=[
            pl.BlockSpec((NC, _R, 128), lambda i: (0, i, 0)),
            pl.BlockSpec((_R, D_H2), lambda i: (i, 0)),
        ],
        out_shape=[
            jax.ShapeDtypeStruct((NC, N_NODES, 128), jnp.float32),
            jax.ShapeDtypeStruct((N_NODES, D_H2), jnp.float32),
        ],
        compiler_params=pltpu.CompilerParams(
            dimension_semantics=("parallel",)),
    )(agg1h, x, w1rT, w1sT, b1r, w2rT, w2sT)


def _dense_c(agg2h, r2, b2r, w3rT, w3sT):
    return pl.pallas_call(
        _c_body,
        grid=(_G,),
        in_specs=[
            pl.BlockSpec((NC, _R, 128), lambda i: (0, i, 0)),
            pl.BlockSpec((_R, D_H2), lambda i: (i, 0)),
            pl.BlockSpec((1, D_H2), lambda i: (0, 0)),
            pl.BlockSpec((D_H2, 2), lambda i: (0, 0)),
            pl.BlockSpec((D_H2, 2), lambda i: (0, 0)),
        ],
        out_specs=[
            pl.BlockSpec((_R,), lambda i: (i,)),
            pl.BlockSpec((_R,), lambda i: (i,)),
            pl.BlockSpec((_R,), lambda i: (i,)),
            pl.BlockSpec((_R,), lambda i: (i,)),
        ],
        out_shape=[
            jax.ShapeDtypeStruct((N_NODES,), jnp.float32),
            jax.ShapeDtypeStruct((N_NODES,), jnp.float32),
            jax.ShapeDtypeStruct((N_NODES,), jnp.float32),
            jax.ShapeDtypeStruct((N_NODES,), jnp.float32),
        ],
        compiler_params=pltpu.CompilerParams(
            dimension_semantics=("parallel",)),
    )(agg2h, r2, b2r, w3rT, w3sT)


def _dense_e(partf, r3c0, r3c1, b3):
    return pl.pallas_call(
        _e_body,
        grid=(1,),
        in_specs=[
            pl.BlockSpec((4 * N_ACC,), lambda i: (0,)),
            pl.BlockSpec((N_NODES,), lambda i: (0,)),
            pl.BlockSpec((N_NODES,), lambda i: (0,)),
            pl.BlockSpec((2,), lambda i: (0,)),
        ],
        out_specs=[
            pl.BlockSpec((N_NODES,), lambda i: (0,)),
            pl.BlockSpec((N_NODES,), lambda i: (0,)),
        ],
        out_shape=[
            jax.ShapeDtypeStruct((N_NODES,), jnp.float32),
            jax.ShapeDtypeStruct((N_NODES,), jnp.float32),
        ],
        compiler_params=pltpu.CompilerParams(
            dimension_semantics=("arbitrary",)),
    )(partf, r3c0, r3c1, b3)


def kernel(x, edge_index, W1_rel, b1, W1_root, W2_rel, b2, W2_root, W3_rel,
           b3, W3_root):
    src = edge_index[0].astype(jnp.int32)
    dst = edge_index[1].astype(jnp.int32)
    srcp = jnp.concatenate([src, jnp.zeros((EP - N_EDGES,), jnp.int32)])
    dstp = jnp.concatenate(
        [dst, jnp.full((EP - N_EDGES,), N_NODES, jnp.int32)])
    srcr16 = srcp.reshape(NS, NCH16, CHUNK)
    dstr16 = dstp.reshape(NS, NCH16, CHUNK)
    srcr32 = srcp.reshape(NC * NS, NCH32, CHUNK)
    dstr32 = dstp.reshape(NC * NS, NCH32, CHUNK)
    z128 = jnp.zeros((N_ACC, 128), jnp.float32)
    z1 = jnp.zeros((N_ACC,), jnp.float32)

    xh = jnp.stack([x[:, :128], x[:, 128:]])
    w1rT = W1_rel.T
    w1sT = W1_root.T
    w2rT = W2_rel.T
    w2sT = W2_root.T
    w3rT = W3_rel.T
    w3sT = W3_root.T
    b1r = b1.reshape(1, D_H1)
    b2r = b2.reshape(1, D_H2)

    agg1h = _segsum_feat(xh, srcr16, dstr16, z128)
    y2h, r2 = _dense_b(agg1h, x, w1rT, w1sT, b1r, w2rT, w2sT)
    agg2h = _segsum_feat(y2h, srcr16, dstr16, z128)
    y3c0, y3c1, r3c0, r3c1 = _dense_c(agg2h, r2, b2r, w3rT, w3sT)
    partf = _segsum_edge(y3c0, y3c1, srcr32, dstr32, z1)
    o0, o1 = _dense_e(partf, r3c0, r3c1, b3)
    return jnp.stack([o0, o1], axis=1)


# speedup vs baseline: 5.6818x; 1.0556x over previous
import functools

import jax
import jax.numpy as jnp
from jax import lax
from jax.experimental import pallas as pl
from jax.experimental.pallas import tpu as pltpu
from jax.experimental.pallas import tpu_sc as plsc

N_NODES = 10000
N_EDGES = 160000
D_IN = 256
D_H1 = 512
D_H2 = 256

NC = 2
NS = 16
CHUNK = 128

EP = 163840
N_ACC = 10112
NCH16 = EP // (NS * CHUNK)
NCH32 = EP // (NC * NS * CHUNK)

def _make_segsum(feat, nch, per_core_edges, tc_tiling=None):
    mesh = plsc.VectorSubcoreMesh(
        core_axis_name="c", subcore_axis_name="s",
        num_cores=NC, num_subcores=NS)

    G = 8
    ng = nch // G
    cparams = (None if tc_tiling is None else
               pltpu.CompilerParams(use_tc_tiling_on_sc=tc_tiling))

    @functools.partial(
        pl.kernel,
        mesh=mesh,
        compiler_params=cparams,
        out_type=jax.ShapeDtypeStruct((NC, N_ACC, feat), jnp.float32),
        scratch_types=[
            pltpu.VMEM_SHARED((N_ACC, feat), jnp.float32),
            pltpu.VMEM((2, G, CHUNK), jnp.int32),
            pltpu.VMEM((2, G, CHUNK), jnp.int32),
            pltpu.VMEM((CHUNK, feat), jnp.float32),
            pltpu.VMEM((CHUNK, feat), jnp.float32),
            pltpu.SemaphoreType.DMA,
            pltpu.SemaphoreType.DMA,
            pltpu.SemaphoreType.DMA,
            pltpu.SemaphoreType.DMA,
        ],
    )
    def seg(tbl, srcr, dstr, zr, out, acc, sv, dv, rb0, rb1,
            sm0, sm1, im0, im1):
        c = lax.axis_index("c")
        s = lax.axis_index("s")
        zrows = N_ACC // NS
        pltpu.sync_copy(zr.at[pl.ds(s * zrows, zrows)],
                        acc.at[pl.ds(s * zrows, zrows)])
        eslice = c * NS + s if per_core_edges else s
        srcrt = srcr.at[eslice]
        dstrt = dstr.at[eslice]

        tblc = tbl if per_core_edges else tbl.at[c]
        bufs = (rb0, rb1)
        sems = (sm0, sm1)
        isems = (im0, im1)

        pltpu.async_copy(srcrt.at[pl.ds(0, G)], sv.at[0], im0)
        pltpu.async_copy(dstrt.at[pl.ds(0, G)], dv.at[0], im0)
        plsc.subcore_barrier()

        @pl.loop(0, ng // 2)
        def _(h):
            for q in range(2):
                g = 2 * h + q
                svq = sv.at[q]
                dvq = dv.at[q]
                pltpu.make_async_copy(srcrt.at[pl.ds(0, G)], svq,
                                      isems[q]).wait()
                pltpu.make_async_copy(dstrt.at[pl.ds(0, G)], dvq,
                                      isems[q]).wait()

                @pl.when(g + 1 < ng)
                def _pref():
                    pltpu.async_copy(srcrt.at[pl.ds((g + 1) * G, G)],
                                     sv.at[1 - q], isems[1 - q])
                    pltpu.async_copy(dstrt.at[pl.ds((g + 1) * G, G)],
                                     dv.at[1 - q], isems[1 - q])

                pltpu.async_copy(tblc.at[svq.at[0]], rb0, sm0)
                pltpu.async_copy(tblc.at[svq.at[1]], rb1, sm1)
                for k in range(G):
                    b = k % 2
                    pltpu.make_async_copy(tblc.at[svq.at[k]], bufs[b],
                                          sems[b]).wait()
                    if k + 2 < G:
                        pltpu.async_copy(tblc.at[svq.at[k + 2]], bufs[b],
                                         sems[b])
                    pltpu.sync_copy(bufs[b], acc.at[dvq.at[k]], add=True)

        plsc.subcore_barrier()
        pltpu.sync_copy(acc.at[pl.ds(s * zrows, zrows)],
                        out.at[c].at[pl.ds(s * zrows, zrows)])

    return seg


@functools.cache
def _segsum_feat_kernel():
    return _make_segsum(128, NCH16, per_core_edges=False)


@functools.cache
def _segsum_edge_kernel():
    mesh = plsc.VectorSubcoreMesh(
        core_axis_name="c", subcore_axis_name="s",
        num_cores=NC, num_subcores=NS)
    nch = NCH32

    @functools.partial(
        pl.kernel,
        mesh=mesh,
        compiler_params=pltpu.CompilerParams(use_tc_tiling_on_sc=False),
        out_type=jax.ShapeDtypeStruct((4 * N_ACC,), jnp.float32),
        scratch_types=[
            pltpu.VMEM_SHARED((2, N_ACC), jnp.float32),
            pltpu.VMEM((nch, CHUNK), jnp.int32),
            pltpu.VMEM((nch, CHUNK), jnp.int32),
            pltpu.VMEM((2, 2, CHUNK), jnp.float32),
            pltpu.SemaphoreType.DMA,
            pltpu.SemaphoreType.DMA,
            pltpu.SemaphoreType.DMA,
            pltpu.SemaphoreType.DMA,
        ],
    )
    def seg(tbl0, tbl1, srcr, dstr, zr, out, acc, sv, dv, vb,
            sa0, sa1, sb0, sb1):
        c = lax.axis_index("c")
        s = lax.axis_index("s")
        eslice = c * NS + s
        pltpu.sync_copy(srcr.at[eslice], sv)
        pltpu.sync_copy(dstr.at[eslice], dv)
        zrows = N_ACC // NS
        pltpu.sync_copy(zr.at[pl.ds(s * zrows, zrows)],
                        acc.at[0].at[pl.ds(s * zrows, zrows)])
        pltpu.sync_copy(zr.at[pl.ds(s * zrows, zrows)],
                        acc.at[1].at[pl.ds(s * zrows, zrows)])
        plsc.subcore_barrier()

        tbls = (tbl0, tbl1)
        sems = ((sa0, sa1), (sb0, sb1))
        for q in range(2):
            for col in range(2):
                pltpu.async_copy(tbls[col].at[sv.at[q]],
                                 vb.at[q].at[col], sems[q][col])

        @pl.loop(0, nch // 2)
        def _(h):
            for q in range(2):
                j = 2 * h + q
                for col in range(2):
                    pltpu.make_async_copy(tbls[col].at[sv.at[j]],
                                          vb.at[q].at[col],
                                          sems[q][col]).wait()
                    pltpu.sync_copy(vb.at[q].at[col],
                                    acc.at[col].at[dv.at[j]], add=True)

                    @pl.when(j + 2 < nch)
                    def _issue():
                        pltpu.async_copy(tbls[col].at[sv.at[j + 2]],
                                         vb.at[q].at[col], sems[q][col])

        plsc.subcore_barrier()
        for col in range(2):
            base = c * (2 * N_ACC) + col * N_ACC + s * zrows
            pltpu.sync_copy(acc.at[col].at[pl.ds(s * zrows, zrows)],
                            out.at[pl.ds(base, zrows)])

    return seg


def _segsum_feat(*args):
    return _segsum_feat_kernel()(*args)


def _segsum_edge(*args):
    return _segsum_edge_kernel()(*args)

_R = 1024
_G = -(-N_NODES // _R)


def _b_body(aggh, x, w1r, w1s, b1, w2r, w2s, y2h, r2):
    agg = jnp.concatenate([aggh[0], aggh[1]], axis=-1)
    h = jnp.dot(agg, w1r[...], preferred_element_type=jnp.float32)
    h = h + jnp.dot(x[...], w1s[...], preferred_element_type=jnp.float32)
    x1 = jnp.maximum(h + b1[...], 0.0)
    y2 = jnp.dot(x1, w2r[...], preferred_element_type=jnp.float32)
    y2h[0] = y2[:, :128]
    y2h[1] = y2[:, 128:]
    r2[...] = jnp.dot(x1, w2s[...], preferred_element_type=jnp.float32)


def _c_body(aggh, r2, b2, w3r, w3s, y3c0, y3c1, r3c0, r3c1):
    x2 = jnp.concatenate([aggh[0], aggh[1]], axis=-1) + r2[...] + b2[...]
    y3 = jnp.dot(x2, w3r[...], preferred_element_type=jnp.float32)
    r3 = jnp.dot(x2, w3s[...], preferred_element_type=jnp.float32)
    y3c0[...] = y3[:, 0]
    y3c1[...] = y3[:, 1]
    r3c0[...] = r3[:, 0]
    r3c1[...] = r3[:, 1]


def _e_body(partf, r3c0, r3c1, b3, out):
    l0 = (partf[0:N_NODES] + partf[2 * N_ACC:2 * N_ACC + N_NODES]
          + r3c0[...] + b3[0:1])
    l1 = (partf[N_ACC:N_ACC + N_NODES]
          + partf[3 * N_ACC:3 * N_ACC + N_NODES] + r3c1[...] + b3[1:2])
    m = jnp.maximum(l0, l1)
    e0 = jnp.exp(l0 - m)
    e1 = jnp.exp(l1 - m)
    inv = 1.0 / (e0 + e1)
    out[...] = jnp.stack([e0 * inv, e1 * inv], axis=1)


def _dense_b(agg1h, x, w1rT, w1sT, b1r, w2rT, w2sT):
    return pl.pallas_call(
        _b_body,
        grid=(_G,),
        in_specs=[
            pl.BlockSpec((NC, _R, 128), lambda i: (0, i, 0)),
            pl.BlockSpec((_R, D_IN), lambda i: (i, 0)),
            pl.BlockSpec((D_IN, D_H1), lambda i: (0, 0)),
            pl.BlockSpec((D_IN, D_H1), lambda i: (0, 0)),
            pl.BlockSpec((1, D_H1), lambda i: (0, 0)),
            pl.BlockSpec((D_H1, D_H2), lambda i: (0, 0)),
            pl.BlockSpec((D_H1, D_H2), lambda i: (0, 0)),
        ],
        out_specs=[
            pl.BlockSpec((NC, _R, 128), lambda i: (0, i, 0)),
            pl.BlockSpec((_R, D_H2), lambda i: (i, 0)),
        ],
        out_shape=[
            jax.ShapeDtypeStruct((NC, N_NODES, 128), jnp.float32),
            jax.ShapeDtypeStruct((N_NODES, D_H2), jnp.float32),
        ],
        compiler_params=pltpu.CompilerParams(
            dimension_semantics=("parallel",)),
    )(agg1h, x, w1rT, w1sT, b1r, w2rT, w2sT)


def _dense_c(agg2h, r2, b2r, w3rT, w3sT):
    return pl.pallas_call(
        _c_body,
        grid=(_G,),
        in_specs=[
            pl.BlockSpec((NC, _R, 128), lambda i: (0, i, 0)),
            pl.BlockSpec((_R, D_H2), lambda i: (i, 0)),
            pl.BlockSpec((1, D_H2), lambda i: (0, 0)),
            pl.BlockSpec((D_H2, 2), lambda i: (0, 0)),
            pl.BlockSpec((D_H2, 2), lambda i: (0, 0)),
        ],
        out_specs=[
            pl.BlockSpec((_R,), lambda i: (i,)),
            pl.BlockSpec((_R,), lambda i: (i,)),
            pl.BlockSpec((_R,), lambda i: (i,)),
            pl.BlockSpec((_R,), lambda i: (i,)),
        ],
        out_shape=[
            jax.ShapeDtypeStruct((N_NODES,), jnp.float32),
            jax.ShapeDtypeStruct((N_NODES,), jnp.float32),
            jax.ShapeDtypeStruct((N_NODES,), jnp.float32),
            jax.ShapeDtypeStruct((N_NODES,), jnp.float32),
        ],
        compiler_params=pltpu.CompilerParams(
            dimension_semantics=("parallel",)),
    )(agg2h, r2, b2r, w3rT, w3sT)


def _dense_e(partf, r3c0, r3c1, b3):
    return pl.pallas_call(
        _e_body,
        grid=(1,),
        in_specs=[
            pl.BlockSpec((4 * N_ACC,), lambda i: (0,)),
            pl.BlockSpec((N_NODES,), lambda i: (0,)),
            pl.BlockSpec((N_NODES,), lambda i: (0,)),
            pl.BlockSpec((2,), lambda i: (0,)),
        ],
        out_specs=pl.BlockSpec((N_NODES, 2), lambda i: (0, 0)),
        out_shape=jax.ShapeDtypeStruct((N_NODES, 2), jnp.float32),
        compiler_params=pltpu.CompilerParams(
            dimension_semantics=("arbitrary",)),
    )(partf, r3c0, r3c1, b3)


def kernel(x, edge_index, W1_rel, b1, W1_root, W2_rel, b2, W2_root, W3_rel,
           b3, W3_root):
    src = edge_index[0].astype(jnp.int32)
    dst = edge_index[1].astype(jnp.int32)
    srcp = jnp.concatenate([src, jnp.zeros((EP - N_EDGES,), jnp.int32)])
    dstp = jnp.concatenate(
        [dst, jnp.full((EP - N_EDGES,), N_NODES, jnp.int32)])
    srcr16 = srcp.reshape(NS, NCH16, CHUNK)
    dstr16 = dstp.reshape(NS, NCH16, CHUNK)
    srcr32 = srcp.reshape(NC * NS, NCH32, CHUNK)
    dstr32 = dstp.reshape(NC * NS, NCH32, CHUNK)
    z128 = jnp.zeros((N_ACC, 128), jnp.float32)
    z1 = jnp.zeros((N_ACC,), jnp.float32)

    xh = jnp.stack([x[:, :128], x[:, 128:]])
    w1rT = W1_rel.T
    w1sT = W1_root.T
    w2rT = W2_rel.T
    w2sT = W2_root.T
    w3rT = W3_rel.T
    w3sT = W3_root.T
    b1r = b1.reshape(1, D_H1)
    b2r = b2.reshape(1, D_H2)

    agg1h = _segsum_feat(xh, srcr16, dstr16, z128)
    y2h, r2 = _dense_b(agg1h, x, w1rT, w1sT, b1r, w2rT, w2sT)
    agg2h = _segsum_feat(y2h, srcr16, dstr16, z128)
    y3c0, y3c1, r3c0, r3c1 = _dense_c(agg2h, r2, b2r, w3rT, w3sT)
    partf = _segsum_edge(y3c0, y3c1, srcr32, dstr32, z1)
    return _dense_e(partf, r3c0, r3c1, b3)
